# baseline (speedup 1.0000x reference)
import functools

import jax
import jax.numpy as jnp
from jax import lax
from jax.experimental import pallas as pl
from jax.experimental.pallas import tpu as pltpu
from jax.experimental.pallas import tpu_sc as plsc

_NC = 2
_NS = 16
_NW = _NC * _NS
_CH = 128


def _make_gather(n, v, d, n_ch):
    mesh = plsc.VectorSubcoreMesh(core_axis_name="c", subcore_axis_name="s")
    rows_per_w = n // _NW

    @functools.partial(
        pl.kernel,
        mesh=mesh,
        compiler_params=pltpu.CompilerParams(use_tc_tiling_on_sc=False),
        out_type=jax.ShapeDtypeStruct((n, d), jnp.float32),
        scratch_types=[
            pltpu.VMEM((n_ch, _CH), jnp.int32),
            pltpu.VMEM((_CH, d), jnp.float32),
            pltpu.SemaphoreType.DMA,
        ],
    )
    def gather_kernel(x_hbm, table_hbm, out_hbm, idx_v, rows_v, sem):
        wid = lax.axis_index("s") * _NC + lax.axis_index("c")
        base = wid * rows_per_w
        pltpu.sync_copy(x_hbm.at[wid], idx_v)

        def body(j, carry):
            pltpu.async_copy(table_hbm.at[idx_v.at[j]], rows_v, sem).wait()
            pltpu.sync_copy(rows_v, out_hbm.at[pl.ds(base + j * _CH, _CH)])
            return carry

        lax.fori_loop(0, n_ch, body, 0)

    return gather_kernel


def kernel(x, table):
    b0, s = x.shape
    v, d = table.shape
    n = b0 * s
    n_ch = n // (_NW * _CH)
    x_blocked = x.reshape(_NW, n_ch, _CH)
    out = _make_gather(n, v, d, n_ch)(x_blocked, table)
    return out.reshape(b0, s, d)


# speedup vs baseline: 1.0445x; 1.0445x over previous
import functools

import jax
import jax.numpy as jnp
from jax import lax
from jax.experimental import pallas as pl
from jax.experimental.pallas import tpu as pltpu
from jax.experimental.pallas import tpu_sc as plsc

_NC = 2
_NS = 16
_NW = _NC * _NS
_CH = 128
_NBUF = 6
_HD = 3


def _make_gather(n, d, n_ch):
    mesh = plsc.VectorSubcoreMesh(core_axis_name="c", subcore_axis_name="s")
    rows_per_w = n // _NW

    @functools.partial(
        pl.kernel,
        mesh=mesh,
        compiler_params=pltpu.CompilerParams(use_tc_tiling_on_sc=False),
        out_type=jax.ShapeDtypeStruct((n, d), jnp.float32),
        scratch_types=[
            pltpu.VMEM((n_ch, _CH), jnp.int32),
            pltpu.VMEM((_NBUF, _CH, d), jnp.float32),
            pltpu.SemaphoreType.DMA,
            pltpu.SemaphoreType.DMA,
        ],
    )
    def gather_kernel(x_hbm, table_hbm, out_hbm, idx_v, rows_v, gsem, ssem):
        wid = lax.axis_index("s") * _NC + lax.axis_index("c")
        base = wid * rows_per_w
        pltpu.sync_copy(x_hbm.at[wid], idx_v)

        def issue_gather(chunk, slot):
            pltpu.async_copy(table_hbm.at[idx_v.at[chunk]], rows_v.at[slot], gsem)

        def wait_gather(slot):
            pltpu.make_async_copy(
                table_hbm.at[idx_v.at[0]], rows_v.at[slot], gsem
            ).wait()

        def issue_scatter(chunk, slot):
            pltpu.async_copy(
                rows_v.at[slot], out_hbm.at[pl.ds(base + chunk * _CH, _CH)], ssem
            )

        def wait_scatter(slot):
            pltpu.make_async_copy(
                rows_v.at[slot], out_hbm.at[pl.ds(base, _CH)], ssem
            ).wait()

        for i in range(_HD):
            issue_gather(i, i)

        for i in range(_NBUF - _HD):
            issue_gather(i + _HD, i + _HD)
            wait_gather(i)
            issue_scatter(i, i)

        def body(i, carry):
            ip = i + _HD
            bp = lax.rem(ip, _NBUF)
            b = lax.rem(i, _NBUF)
            wait_scatter(bp)
            issue_gather(ip, bp)
            wait_gather(b)
            issue_scatter(i, b)
            return carry

        lax.fori_loop(_NBUF - _HD, n_ch - _HD, body, 0)

        for i in range(n_ch - _HD, n_ch):
            b = i % _NBUF
            wait_gather(b)
            issue_scatter(i, b)
        for _ in range(_NBUF):
            wait_scatter(0)

    return gather_kernel


def kernel(x, table):
    b0, s = x.shape
    v, d = table.shape
    n = b0 * s
    n_ch = n // (_NW * _CH)
    x_blocked = x.reshape(_NW, n_ch, _CH)
    out = _make_gather(n, d, n_ch)(x_blocked, table)
    return out.reshape(b0, s, d)
